# baseline (speedup 1.0000x reference)
import functools
import jax
import jax.numpy as jnp
from jax import lax
from jax.experimental import pallas as pl
from jax.experimental.pallas import tpu as pltpu
from jax.experimental.pallas import tpu_sc as plsc

NC, NS, L = 2, 16, 16
NW = NC * NS
CH = 2048

LN2 = 0.6931471805599453
SQRT2 = 1.4142135623730951


def _tree(op, xs):
    while len(xs) > 1:
        nxt = [op(xs[i], xs[i + 1]) for i in range(0, len(xs) - 1, 2)]
        if len(xs) % 2:
            nxt.append(xs[-1])
        xs = nxt
    return xs[0]


def _sc_body(ncls, M, HW, W, PW, out_hbm, lbl_hbm, lw_hbm, sfp_hbm, w_hbm,
             sum_out, cnt_out,
             lbuf0, s0buf0, s1buf0, lblbuf0, lwbuf0,
             lbuf1, s0buf1, s1buf1, lblbuf1, lwbuf1,
             idxbuf, wbuf, accv, cntv, sem0, sem1):
    cid = lax.axis_index("c")
    sid = lax.axis_index("s")
    wid = sid * NC + cid
    base = wid * PW
    b = base // M
    moff = base - b * M
    R = CH // W
    wsh = W.bit_length() - 1

    pltpu.sync_copy(w_hbm, wbuf)

    acc = jnp.zeros((L,), jnp.float32)
    tot = jnp.int32(0)

    nch = PW // CH
    nv = CH // L
    lane = lax.iota(jnp.int32, L)
    wq = jnp.right_shift(lane, 2)
    bsh = jnp.bitwise_and(lane, 3) * 8

    nvw = W // L

    def mbody(s0buf, s1buf, lwbuf, i, cnt):
        o16 = i * L
        r = i // nvw
        w0 = (i - r * nvw) * L
        rsl = pl.ds(w0, L)
        s0 = s0buf[r, rsl]
        s1 = s1buf[r, rsl]
        wiv = (o16 // 4) + wq
        word = plsc.load_gather(
            lwbuf, [jnp.right_shift(wiv, wsh), jnp.bitwise_and(wiv, W - 1)])
        bit = jnp.bitwise_and(jnp.right_shift(word, bsh), 1)
        mk = jnp.bitwise_and(jnp.where(s1 > s0, 1, 0), bit) != 0
        pos = o16 + lane
        plsc.store_compressed(idxbuf.at[pl.ds(cnt, L)], pos, mask=mk)
        return cnt + jnp.sum(mk.astype(jnp.int32))

    def cbody(lbuf, lblbuf, ccnt, i, acc):
        o16 = i * L
        posv = idxbuf[pl.ds(o16, L)]
        lanev = lane < (ccnt - o16)
        ph = jnp.right_shift(posv, wsh)
        pw = jnp.bitwise_and(posv, W - 1)
        xs = [plsc.load_gather(lbuf, [jnp.full((L,), c, jnp.int32), ph, pw])
              for c in range(ncls)]
        mx = _tree(jnp.maximum, xs)
        s = _tree(jnp.add, [jnp.exp(x - mx) for x in xs])
        bits = lax.bitcast_convert_type(s, jnp.int32)
        e = jnp.right_shift(bits, 23) - 127
        mant = lax.bitcast_convert_type(
            jnp.bitwise_or(jnp.bitwise_and(bits, 0x7FFFFF), 0x3F800000),
            jnp.float32)
        adj = mant > SQRT2
        mant = jnp.where(adj, mant * 0.5, mant)
        e = e + jnp.where(adj, 1, 0)
        t = (mant - 1.0) / (mant + 1.0)
        t2 = t * t
        logm = 2.0 * t * (1.0 + t2 * (1.0 / 3.0 + t2 * (0.2 + t2 * (1.0 / 7.0))))
        logs = e.astype(jnp.float32) * LN2 + logm

        lblv = plsc.load_gather(lblbuf, [ph, pw])
        validm = lblv != 255
        tgt = jnp.where(validm, lblv, 0)
        xt = plsc.load_gather(lbuf, [tgt, ph, pw])
        wv = plsc.load_gather(wbuf, [tgt]) * jnp.where(validm, 1.0, 0.0)

        nll = logs + mx - xt
        return acc + jnp.where(lanev, nll * wv, 0.0)

    bufs = [
        (lbuf0, s0buf0, s1buf0, lblbuf0, lwbuf0),
        (lbuf1, s0buf1, s1buf1, lblbuf1, lwbuf1),
    ]
    sems = [sem0, sem1]

    def fire(c, si):
        off = moff + c * CH
        d = off // HW
        h0 = (off - d * HW) // W
        lbuf, s0buf, s1buf, lblbuf, lwbuf = bufs[si]
        sem = sems[si]
        pltpu.async_copy(out_hbm.at[b, :, d, pl.ds(h0, R), :], lbuf, sem)
        pltpu.async_copy(sfp_hbm.at[b, 0, d, pl.ds(h0, R), :], s0buf, sem)
        pltpu.async_copy(sfp_hbm.at[b, 1, d, pl.ds(h0, R), :], s1buf, sem)
        pltpu.async_copy(lbl_hbm.at[b, d, pl.ds(h0, R), :], lblbuf, sem)
        pltpu.async_copy(lw_hbm.at[b, pl.ds(off // 512, 4), :], lwbuf, sem)

    def wait_set(si):
        lbuf, s0buf, s1buf, lblbuf, lwbuf = bufs[si]
        sem = sems[si]
        pltpu.make_async_copy(
            out_hbm.at[0, :, 0, pl.ds(0, R), :], lbuf, sem).wait()
        pltpu.make_async_copy(
            sfp_hbm.at[0, 0, 0, pl.ds(0, R), :], s0buf, sem).wait()
        pltpu.make_async_copy(
            sfp_hbm.at[0, 1, 0, pl.ds(0, R), :], s1buf, sem).wait()
        pltpu.make_async_copy(
            lbl_hbm.at[0, 0, pl.ds(0, R), :], lblbuf, sem).wait()
        pltpu.make_async_copy(lw_hbm.at[0, pl.ds(0, 4), :], lwbuf, sem).wait()

    def process(si, acc, tot):
        lbuf, s0buf, s1buf, lblbuf, lwbuf = bufs[si]
        ccnt = plsc.parallel_loop(
            0, nv, 1, unroll=4, carry=jnp.int32(0))(
                functools.partial(mbody, s0buf, s1buf, lwbuf))
        idxbuf[pl.ds(ccnt, L)] = jnp.zeros((L,), jnp.int32)
        nvec = (ccnt + (L - 1)) // L
        acc = plsc.parallel_loop(
            0, nvec, 1, unroll=2, carry=acc)(
                functools.partial(cbody, lbuf, lblbuf, ccnt))
        return acc, tot + ccnt

    fire(0, 0)

    def outer(g, carry):
        acc, tot = carry
        c0 = g * 2
        fire(c0 + 1, 1)
        wait_set(0)
        acc, tot = process(0, acc, tot)

        @pl.when(c0 + 2 < nch)
        def _():
            fire(c0 + 2, 0)

        wait_set(1)
        acc, tot = process(1, acc, tot)
        return acc, tot

    acc, tot = lax.fori_loop(0, nch // 2, outer, (acc, tot))

    accv[...] = acc
    cntv[...] = jnp.where(lane == 0, jnp.full((L,), tot, jnp.int32), 0)
    orow = wid // 4
    ocol = (wid - orow * 4) * L
    pltpu.sync_copy(accv, sum_out.at[orow, pl.ds(ocol, L)])
    pltpu.sync_copy(cntv, cnt_out.at[orow, pl.ds(ocol, L)])


def _tc_reduce_body(sum_ref, cnt_ref, loss_ref, flag_ref):
    s = jnp.sum(sum_ref[:, :4 * L])
    c = jnp.sum(cnt_ref[:, :4 * L])
    denom = jnp.maximum(c.astype(jnp.float32), 1.0)
    loss_ref[...] = jnp.where(c > 0, s / denom, 0.0)
    flag_ref[...] = c > 0


@jax.jit
def kernel(output, label, label_weight, sketch_from_pred, weight):
    B = output.shape[0]
    ncls = output.shape[1]
    D, H, W = output.shape[2], output.shape[3], output.shape[4]
    M = D * H * W
    HW = H * W
    N = B * M
    PW = N // NW
    R = CH // W

    lwp = lax.bitcast_convert_type(
        label_weight.astype(jnp.int8).reshape(B, M // 4, 4), jnp.int32
    ).reshape(B, M // 512, 128)

    mesh = plsc.VectorSubcoreMesh(core_axis_name="c", subcore_axis_name="s")
    sc = pl.kernel(
        functools.partial(_sc_body, ncls, M, HW, W, PW),
        out_type=(
            jax.ShapeDtypeStruct((8, 8 * L), jnp.float32),
            jax.ShapeDtypeStruct((8, 8 * L), jnp.int32),
        ),
        mesh=mesh,
        compiler_params=pltpu.CompilerParams(
            use_tc_tiling_on_sc=False, needs_layout_passes=False),
        scratch_types=[
            pltpu.VMEM((ncls, R, W), jnp.float32),
            pltpu.VMEM((R, W), jnp.float32),
            pltpu.VMEM((R, W), jnp.float32),
            pltpu.VMEM((R, W), jnp.int32),
            pltpu.VMEM((4, W), jnp.int32),
            pltpu.VMEM((ncls, R, W), jnp.float32),
            pltpu.VMEM((R, W), jnp.float32),
            pltpu.VMEM((R, W), jnp.float32),
            pltpu.VMEM((R, W), jnp.int32),
            pltpu.VMEM((4, W), jnp.int32),
            pltpu.VMEM((CH + L,), jnp.int32),
            pltpu.VMEM((ncls,), jnp.float32),
            pltpu.VMEM((L,), jnp.float32),
            pltpu.VMEM((L,), jnp.int32),
            pltpu.SemaphoreType.DMA,
            pltpu.SemaphoreType.DMA,
        ],
    )
    sums, cnts = sc(output, label, lwp, sketch_from_pred, weight)

    edge_loss, has_edges = pl.pallas_call(
        _tc_reduce_body,
        out_shape=(
            jax.ShapeDtypeStruct((), jnp.float32),
            jax.ShapeDtypeStruct((), jnp.bool_),
        ),
        out_specs=(
            pl.BlockSpec(memory_space=pltpu.SMEM),
            pl.BlockSpec(memory_space=pltpu.SMEM),
        ),
    )(sums, cnts)

    return (edge_loss, has_edges)


# speedup vs baseline: 5.1313x; 5.1313x over previous
import functools
import jax
import jax.numpy as jnp
from jax import lax
from jax.experimental import pallas as pl
from jax.experimental.pallas import tpu as pltpu
from jax.experimental.pallas import tpu_sc as plsc

NC, NS, L = 2, 16, 16
NW = NC * NS
CH = 2048

LN2 = 0.6931471805599453
SQRT2 = 1.4142135623730951


def _tree(op, xs):
    while len(xs) > 1:
        nxt = [op(xs[i], xs[i + 1]) for i in range(0, len(xs) - 1, 2)]
        if len(xs) % 2:
            nxt.append(xs[-1])
        xs = nxt
    return xs[0]


def _sc_body(ncls, M, HW, W, PW, out_hbm, lbl_hbm, lw_hbm, sfp_hbm, w_hbm,
             sum_out, cnt_out,
             lbuf0, s0buf0, s1buf0, lblbuf0, lwbuf0,
             lbuf1, s0buf1, s1buf1, lblbuf1, lwbuf1,
             idxbuf, wbuf, accv, cntv, sem0, sem1):
    cid = lax.axis_index("c")
    sid = lax.axis_index("s")
    wid = sid * NC + cid
    base = wid * PW
    b = base // M
    moff = base - b * M
    R = CH // W
    wsh = W.bit_length() - 1

    pltpu.sync_copy(w_hbm, wbuf)

    acc = jnp.zeros((L,), jnp.float32)
    tot = jnp.int32(0)

    nch = PW // CH
    nv = CH // L
    lane = lax.iota(jnp.int32, L)

    nvw = W // L

    def mbody(s0buf, s1buf, lwbuf, i, cnt):
        o16 = i * L
        r = i // nvw
        w0 = (i - r * nvw) * L
        rsl = pl.ds(w0, L)
        s0 = s0buf[r, rsl]
        s1 = s1buf[r, rsl]
        lwv = lwbuf[pl.ds(o16, L)]
        mk = jnp.bitwise_and(jnp.where(s1 > s0, 1, 0), lwv) != 0
        pos = o16 + lane
        plsc.store_compressed(idxbuf.at[pl.ds(cnt, L)], pos, mask=mk)
        return cnt + jnp.sum(mk.astype(jnp.int32))

    def cbody(lbuf, lblbuf, ccnt, i, acc):
        o16 = i * L
        posv = idxbuf[pl.ds(o16, L)]
        lanev = lane < (ccnt - o16)
        ph = jnp.right_shift(posv, wsh)
        pw = jnp.bitwise_and(posv, W - 1)
        xs = [plsc.load_gather(lbuf, [jnp.full((L,), c, jnp.int32), ph, pw])
              for c in range(ncls)]
        mx = _tree(jnp.maximum, xs)
        s = _tree(jnp.add, [jnp.exp(x - mx) for x in xs])
        bits = lax.bitcast_convert_type(s, jnp.int32)
        e = jnp.right_shift(bits, 23) - 127
        mant = lax.bitcast_convert_type(
            jnp.bitwise_or(jnp.bitwise_and(bits, 0x7FFFFF), 0x3F800000),
            jnp.float32)
        adj = mant > SQRT2
        mant = jnp.where(adj, mant * 0.5, mant)
        e = e + jnp.where(adj, 1, 0)
        t = (mant - 1.0) / (mant + 1.0)
        t2 = t * t
        logm = 2.0 * t * (1.0 + t2 * (1.0 / 3.0 + t2 * (0.2 + t2 * (1.0 / 7.0))))
        logs = e.astype(jnp.float32) * LN2 + logm

        lblv = plsc.load_gather(lblbuf, [ph, pw])
        validm = lblv != 255
        tgt = jnp.where(validm, lblv, 0)
        xt = plsc.load_gather(lbuf, [tgt, ph, pw])
        wv = plsc.load_gather(wbuf, [tgt]) * jnp.where(validm, 1.0, 0.0)

        nll = logs + mx - xt
        return acc + jnp.where(lanev, nll * wv, 0.0)

    bufs = [
        (lbuf0, s0buf0, s1buf0, lblbuf0, lwbuf0),
        (lbuf1, s0buf1, s1buf1, lblbuf1, lwbuf1),
    ]
    sems = [sem0, sem1]

    def fire(c, si):
        off = moff + c * CH
        d = off // HW
        h0 = (off - d * HW) // W
        lbuf, s0buf, s1buf, lblbuf, lwbuf = bufs[si]
        sem = sems[si]
        pltpu.async_copy(out_hbm.at[b, :, d, pl.ds(h0, R), :], lbuf, sem)
        pltpu.async_copy(sfp_hbm.at[b, 0, d, pl.ds(h0, R), :], s0buf, sem)
        pltpu.async_copy(sfp_hbm.at[b, 1, d, pl.ds(h0, R), :], s1buf, sem)
        pltpu.async_copy(lbl_hbm.at[b, d, pl.ds(h0, R), :], lblbuf, sem)
        pltpu.async_copy(lw_hbm.at[b, pl.ds(off, CH)], lwbuf, sem)

    def wait_set(si):
        lbuf, s0buf, s1buf, lblbuf, lwbuf = bufs[si]
        sem = sems[si]
        pltpu.make_async_copy(
            out_hbm.at[0, :, 0, pl.ds(0, R), :], lbuf, sem).wait()
        pltpu.make_async_copy(
            sfp_hbm.at[0, 0, 0, pl.ds(0, R), :], s0buf, sem).wait()
        pltpu.make_async_copy(
            sfp_hbm.at[0, 1, 0, pl.ds(0, R), :], s1buf, sem).wait()
        pltpu.make_async_copy(
            lbl_hbm.at[0, 0, pl.ds(0, R), :], lblbuf, sem).wait()
        pltpu.make_async_copy(lw_hbm.at[0, pl.ds(0, CH)], lwbuf, sem).wait()

    def process(si, acc, tot):
        lbuf, s0buf, s1buf, lblbuf, lwbuf = bufs[si]
        ccnt = plsc.parallel_loop(
            0, nv, 1, unroll=4, carry=jnp.int32(0))(
                functools.partial(mbody, s0buf, s1buf, lwbuf))
        idxbuf[pl.ds(ccnt, L)] = jnp.zeros((L,), jnp.int32)
        nvec = (ccnt + (L - 1)) // L
        acc = plsc.parallel_loop(
            0, nvec, 1, unroll=2, carry=acc)(
                functools.partial(cbody, lbuf, lblbuf, ccnt))
        return acc, tot + ccnt

    fire(0, 0)

    def outer(g, carry):
        acc, tot = carry
        c0 = g * 2
        fire(c0 + 1, 1)
        wait_set(0)
        acc, tot = process(0, acc, tot)

        @pl.when(c0 + 2 < nch)
        def _():
            fire(c0 + 2, 0)

        wait_set(1)
        acc, tot = process(1, acc, tot)
        return acc, tot

    acc, tot = lax.fori_loop(0, nch // 2, outer, (acc, tot))

    accv[...] = acc
    cntv[...] = jnp.where(lane == 0, jnp.full((L,), tot, jnp.int32), 0)
    orow = wid // 4
    ocol = (wid - orow * 4) * L
    pltpu.sync_copy(accv, sum_out.at[orow, pl.ds(ocol, L)])
    pltpu.sync_copy(cntv, cnt_out.at[orow, pl.ds(ocol, L)])


def _tc_reduce_body(sum_ref, cnt_ref, loss_ref, flag_ref):
    s = jnp.sum(sum_ref[:, :4 * L])
    c = jnp.sum(cnt_ref[:, :4 * L])
    denom = jnp.maximum(c.astype(jnp.float32), 1.0)
    loss_ref[...] = jnp.where(c > 0, s / denom, 0.0)
    flag_ref[...] = c > 0


@jax.jit
def kernel(output, label, label_weight, sketch_from_pred, weight):
    B = output.shape[0]
    ncls = output.shape[1]
    D, H, W = output.shape[2], output.shape[3], output.shape[4]
    M = D * H * W
    HW = H * W
    N = B * M
    PW = N // NW
    R = CH // W

    mesh = plsc.VectorSubcoreMesh(core_axis_name="c", subcore_axis_name="s")
    sc = pl.kernel(
        functools.partial(_sc_body, ncls, M, HW, W, PW),
        out_type=(
            jax.ShapeDtypeStruct((8, 8 * L), jnp.float32),
            jax.ShapeDtypeStruct((8, 8 * L), jnp.int32),
        ),
        mesh=mesh,
        compiler_params=pltpu.CompilerParams(
            use_tc_tiling_on_sc=False, needs_layout_passes=False),
        scratch_types=[
            pltpu.VMEM((ncls, R, W), jnp.float32),
            pltpu.VMEM((R, W), jnp.float32),
            pltpu.VMEM((R, W), jnp.float32),
            pltpu.VMEM((R, W), jnp.int32),
            pltpu.VMEM((CH,), jnp.int32),
            pltpu.VMEM((ncls, R, W), jnp.float32),
            pltpu.VMEM((R, W), jnp.float32),
            pltpu.VMEM((R, W), jnp.float32),
            pltpu.VMEM((R, W), jnp.int32),
            pltpu.VMEM((CH,), jnp.int32),
            pltpu.VMEM((CH + L,), jnp.int32),
            pltpu.VMEM((ncls,), jnp.float32),
            pltpu.VMEM((L,), jnp.float32),
            pltpu.VMEM((L,), jnp.int32),
            pltpu.SemaphoreType.DMA,
            pltpu.SemaphoreType.DMA,
        ],
    )
    sums, cnts = sc(output, label, label_weight, sketch_from_pred, weight)

    edge_loss, has_edges = pl.pallas_call(
        _tc_reduce_body,
        out_shape=(
            jax.ShapeDtypeStruct((), jnp.float32),
            jax.ShapeDtypeStruct((), jnp.bool_),
        ),
        out_specs=(
            pl.BlockSpec(memory_space=pltpu.SMEM),
            pl.BlockSpec(memory_space=pltpu.SMEM),
        ),
    )(sums, cnts)

    return (edge_loss, has_edges)


# speedup vs baseline: 5.1324x; 1.0002x over previous
import functools
import jax
import jax.numpy as jnp
from jax import lax
from jax.experimental import pallas as pl
from jax.experimental.pallas import tpu as pltpu
from jax.experimental.pallas import tpu_sc as plsc

NC, NS, L = 2, 16, 16
NW = NC * NS
CH = 2048

LN2 = 0.6931471805599453
SQRT2 = 1.4142135623730951


def _tree(op, xs):
    while len(xs) > 1:
        nxt = [op(xs[i], xs[i + 1]) for i in range(0, len(xs) - 1, 2)]
        if len(xs) % 2:
            nxt.append(xs[-1])
        xs = nxt
    return xs[0]


def _sc_body(ncls, M, HW, W, PW, out_hbm, lbl_hbm, lw_hbm, sfp_hbm, w_hbm,
             sum_out, cnt_out,
             lbuf0, s0buf0, s1buf0, lblbuf0, lwbuf0,
             lbuf1, s0buf1, s1buf1, lblbuf1, lwbuf1,
             idxbuf, wbuf, accv, cntv, sem0, sem1):
    cid = lax.axis_index("c")
    sid = lax.axis_index("s")
    wid = sid * NC + cid
    base = wid * PW
    b = base // M
    moff = base - b * M
    R = CH // W
    wsh = W.bit_length() - 1

    pltpu.sync_copy(w_hbm, wbuf)

    acc = jnp.zeros((L,), jnp.float32)
    tot = jnp.int32(0)

    nch = PW // CH
    nv = CH // L
    lane = lax.iota(jnp.int32, L)

    nvw = W // L

    def mbody(s0buf, s1buf, lwbuf, i, cnt):
        o16 = i * L
        r = i // nvw
        w0 = (i - r * nvw) * L
        rsl = pl.ds(w0, L)
        s0 = s0buf[r, rsl]
        s1 = s1buf[r, rsl]
        lwv = lwbuf[pl.ds(o16, L)]
        mk = jnp.bitwise_and(jnp.where(s1 > s0, 1, 0), lwv) != 0
        pos = o16 + lane
        plsc.store_compressed(idxbuf.at[pl.ds(cnt, L)], pos, mask=mk)
        return cnt + jnp.sum(mk.astype(jnp.int32))

    def cbody(lbuf, lblbuf, ccnt, i, acc):
        o16 = i * L
        posv = idxbuf[pl.ds(o16, L)]
        lanev = lane < (ccnt - o16)
        ph = jnp.right_shift(posv, wsh)
        pw = jnp.bitwise_and(posv, W - 1)
        xs = [plsc.load_gather(lbuf, [jnp.full((L,), c, jnp.int32), ph, pw])
              for c in range(ncls)]
        mx = _tree(jnp.maximum, xs)
        s = _tree(jnp.add, [jnp.exp(x - mx) for x in xs])
        bits = lax.bitcast_convert_type(s, jnp.int32)
        e = jnp.right_shift(bits, 23) - 127
        mant = lax.bitcast_convert_type(
            jnp.bitwise_or(jnp.bitwise_and(bits, 0x7FFFFF), 0x3F800000),
            jnp.float32)
        adj = mant > SQRT2
        mant = jnp.where(adj, mant * 0.5, mant)
        e = e + jnp.where(adj, 1, 0)
        t = (mant - 1.0) / (mant + 1.0)
        t2 = t * t
        logm = 2.0 * t * (1.0 + t2 * (1.0 / 3.0 + t2 * (0.2 + t2 * (1.0 / 7.0))))
        logs = e.astype(jnp.float32) * LN2 + logm

        lblv = plsc.load_gather(lblbuf, [ph, pw])
        validm = lblv != 255
        tgt = jnp.where(validm, lblv, 0)
        xt = plsc.load_gather(lbuf, [tgt, ph, pw])
        wv = plsc.load_gather(wbuf, [tgt]) * jnp.where(validm, 1.0, 0.0)

        nll = logs + mx - xt
        return acc + jnp.where(lanev, nll * wv, 0.0)

    bufs = [
        (lbuf0, s0buf0, s1buf0, lblbuf0, lwbuf0),
        (lbuf1, s0buf1, s1buf1, lblbuf1, lwbuf1),
    ]
    sems = [sem0, sem1]

    def fire(c, si):
        off = moff + c * CH
        d = off // HW
        h0 = (off - d * HW) // W
        lbuf, s0buf, s1buf, lblbuf, lwbuf = bufs[si]
        sem = sems[si]
        pltpu.async_copy(out_hbm.at[b, :, d, pl.ds(h0, R), :], lbuf, sem)
        pltpu.async_copy(sfp_hbm.at[b, 0, d, pl.ds(h0, R), :], s0buf, sem)
        pltpu.async_copy(sfp_hbm.at[b, 1, d, pl.ds(h0, R), :], s1buf, sem)
        pltpu.async_copy(lbl_hbm.at[b, d, pl.ds(h0, R), :], lblbuf, sem)
        pltpu.async_copy(lw_hbm.at[b, pl.ds(off, CH)], lwbuf, sem)

    def wait_set(si):
        lbuf, s0buf, s1buf, lblbuf, lwbuf = bufs[si]
        sem = sems[si]
        pltpu.make_async_copy(
            out_hbm.at[0, :, 0, pl.ds(0, R), :], lbuf, sem).wait()
        pltpu.make_async_copy(
            sfp_hbm.at[0, 0, 0, pl.ds(0, R), :], s0buf, sem).wait()
        pltpu.make_async_copy(
            sfp_hbm.at[0, 1, 0, pl.ds(0, R), :], s1buf, sem).wait()
        pltpu.make_async_copy(
            lbl_hbm.at[0, 0, pl.ds(0, R), :], lblbuf, sem).wait()
        pltpu.make_async_copy(lw_hbm.at[0, pl.ds(0, CH)], lwbuf, sem).wait()

    def process(si, acc, tot):
        lbuf, s0buf, s1buf, lblbuf, lwbuf = bufs[si]
        ccnt = plsc.parallel_loop(
            0, nv, 1, unroll=4, carry=jnp.int32(0))(
                functools.partial(mbody, s0buf, s1buf, lwbuf))
        idxbuf[pl.ds(ccnt, L)] = jnp.zeros((L,), jnp.int32)
        nvec = (ccnt + (L - 1)) // L
        acc = plsc.parallel_loop(
            0, nvec, 1, unroll=3, carry=acc)(
                functools.partial(cbody, lbuf, lblbuf, ccnt))
        return acc, tot + ccnt

    fire(0, 0)

    def outer(g, carry):
        acc, tot = carry
        c0 = g * 2
        fire(c0 + 1, 1)
        wait_set(0)
        acc, tot = process(0, acc, tot)

        @pl.when(c0 + 2 < nch)
        def _():
            fire(c0 + 2, 0)

        wait_set(1)
        acc, tot = process(1, acc, tot)
        return acc, tot

    acc, tot = lax.fori_loop(0, nch // 2, outer, (acc, tot))

    accv[...] = acc
    cntv[...] = jnp.where(lane == 0, jnp.full((L,), tot, jnp.int32), 0)
    orow = wid // 4
    ocol = (wid - orow * 4) * L
    pltpu.sync_copy(accv, sum_out.at[orow, pl.ds(ocol, L)])
    pltpu.sync_copy(cntv, cnt_out.at[orow, pl.ds(ocol, L)])


def _tc_reduce_body(sum_ref, cnt_ref, loss_ref, flag_ref):
    s = jnp.sum(sum_ref[:, :4 * L])
    c = jnp.sum(cnt_ref[:, :4 * L])
    denom = jnp.maximum(c.astype(jnp.float32), 1.0)
    loss_ref[...] = jnp.where(c > 0, s / denom, 0.0)
    flag_ref[...] = c > 0


@jax.jit
def kernel(output, label, label_weight, sketch_from_pred, weight):
    B = output.shape[0]
    ncls = output.shape[1]
    D, H, W = output.shape[2], output.shape[3], output.shape[4]
    M = D * H * W
    HW = H * W
    N = B * M
    PW = N // NW
    R = CH // W

    mesh = plsc.VectorSubcoreMesh(core_axis_name="c", subcore_axis_name="s")
    sc = pl.kernel(
        functools.partial(_sc_body, ncls, M, HW, W, PW),
        out_type=(
            jax.ShapeDtypeStruct((8, 8 * L), jnp.float32),
            jax.ShapeDtypeStruct((8, 8 * L), jnp.int32),
        ),
        mesh=mesh,
        compiler_params=pltpu.CompilerParams(
            use_tc_tiling_on_sc=False, needs_layout_passes=False),
        scratch_types=[
            pltpu.VMEM((ncls, R, W), jnp.float32),
            pltpu.VMEM((R, W), jnp.float32),
            pltpu.VMEM((R, W), jnp.float32),
            pltpu.VMEM((R, W), jnp.int32),
            pltpu.VMEM((CH,), jnp.int32),
            pltpu.VMEM((ncls, R, W), jnp.float32),
            pltpu.VMEM((R, W), jnp.float32),
            pltpu.VMEM((R, W), jnp.float32),
            pltpu.VMEM((R, W), jnp.int32),
            pltpu.VMEM((CH,), jnp.int32),
            pltpu.VMEM((CH + L,), jnp.int32),
            pltpu.VMEM((ncls,), jnp.float32),
            pltpu.VMEM((L,), jnp.float32),
            pltpu.VMEM((L,), jnp.int32),
            pltpu.SemaphoreType.DMA,
            pltpu.SemaphoreType.DMA,
        ],
    )
    sums, cnts = sc(output, label, label_weight, sketch_from_pred, weight)

    edge_loss, has_edges = pl.pallas_call(
        _tc_reduce_body,
        out_shape=(
            jax.ShapeDtypeStruct((), jnp.float32),
            jax.ShapeDtypeStruct((), jnp.bool_),
        ),
        out_specs=(
            pl.BlockSpec(memory_space=pltpu.SMEM),
            pl.BlockSpec(memory_space=pltpu.SMEM),
        ),
    )(sums, cnts)

    return (edge_loss, has_edges)


# speedup vs baseline: 5.1366x; 1.0008x over previous
import functools
import jax
import jax.numpy as jnp
from jax import lax
from jax.experimental import pallas as pl
from jax.experimental.pallas import tpu as pltpu
from jax.experimental.pallas import tpu_sc as plsc

NC, NS, L = 2, 16, 16
NW = NC * NS
CH = 2048

LN2 = 0.6931471805599453
SQRT2 = 1.4142135623730951


def _tree(op, xs):
    while len(xs) > 1:
        nxt = [op(xs[i], xs[i + 1]) for i in range(0, len(xs) - 1, 2)]
        if len(xs) % 2:
            nxt.append(xs[-1])
        xs = nxt
    return xs[0]


def _sc_body(ncls, M, HW, W, PW, out_hbm, lbl_hbm, lw_hbm, sfp_hbm, w_hbm,
             sum_out, cnt_out,
             lbuf0, s0buf0, s1buf0, lblbuf0, lwbuf0,
             lbuf1, s0buf1, s1buf1, lblbuf1, lwbuf1,
             idxbuf, wbuf, accv, cntv, sem0, sem1):
    cid = lax.axis_index("c")
    sid = lax.axis_index("s")
    wid = sid * NC + cid
    base = wid * PW
    b = base // M
    moff = base - b * M
    R = CH // W
    wsh = W.bit_length() - 1

    pltpu.sync_copy(w_hbm, wbuf)

    acc = jnp.zeros((L,), jnp.float32)
    tot = jnp.int32(0)

    nch = PW // CH
    nv = CH // L
    lane = lax.iota(jnp.int32, L)

    nvw = W // L

    def mbody(s0buf, s1buf, lwbuf, i, cnt):
        o16 = i * L
        r = i // nvw
        w0 = (i - r * nvw) * L
        rsl = pl.ds(w0, L)
        s0 = s0buf[r, rsl]
        s1 = s1buf[r, rsl]
        lwv = lwbuf[pl.ds(o16, L)]
        mk = jnp.bitwise_and(jnp.where(s1 > s0, 1, 0), lwv) != 0
        pos = o16 + lane
        plsc.store_compressed(idxbuf.at[pl.ds(cnt, L)], pos, mask=mk)
        return cnt + jnp.sum(mk.astype(jnp.int32))

    def cbody(lbuf, lblbuf, ccnt, i, acc):
        o16 = i * L
        posv = idxbuf[pl.ds(o16, L)]
        lanev = lane < (ccnt - o16)
        ph = jnp.right_shift(posv, wsh)
        pw = jnp.bitwise_and(posv, W - 1)
        xs = [plsc.load_gather(lbuf, [jnp.full((L,), c, jnp.int32), ph, pw])
              for c in range(ncls)]
        mx = _tree(jnp.maximum, xs)
        s = _tree(jnp.add, [jnp.exp(x - mx) for x in xs])
        bits = lax.bitcast_convert_type(s, jnp.int32)
        e = jnp.right_shift(bits, 23) - 127
        mant = lax.bitcast_convert_type(
            jnp.bitwise_or(jnp.bitwise_and(bits, 0x7FFFFF), 0x3F800000),
            jnp.float32)
        adj = mant > SQRT2
        mant = jnp.where(adj, mant * 0.5, mant)
        e = e + jnp.where(adj, 1, 0)
        t = (mant - 1.0) / (mant + 1.0)
        t2 = t * t
        logm = 2.0 * t * (1.0 + t2 * (1.0 / 3.0 + t2 * (0.2 + t2 * (1.0 / 7.0))))
        logs = e.astype(jnp.float32) * LN2 + logm

        lblv = plsc.load_gather(lblbuf, [ph, pw])
        validm = lblv != 255
        tgt = jnp.where(validm, lblv, 0)
        xt = plsc.load_gather(lbuf, [tgt, ph, pw])
        wv = plsc.load_gather(wbuf, [tgt]) * jnp.where(validm, 1.0, 0.0)

        nll = logs + mx - xt
        return acc + jnp.where(lanev, nll * wv, 0.0)

    bufs = [
        (lbuf0, s0buf0, s1buf0, lblbuf0, lwbuf0),
        (lbuf1, s0buf1, s1buf1, lblbuf1, lwbuf1),
    ]
    sems = [sem0, sem1]

    def fire(c, si):
        off = moff + c * CH
        d = off // HW
        h0 = (off - d * HW) // W
        lbuf, s0buf, s1buf, lblbuf, lwbuf = bufs[si]
        sem = sems[si]
        pltpu.async_copy(out_hbm.at[b, :, d, pl.ds(h0, R), :], lbuf, sem)
        pltpu.async_copy(sfp_hbm.at[b, 0, d, pl.ds(h0, R), :], s0buf, sem)
        pltpu.async_copy(sfp_hbm.at[b, 1, d, pl.ds(h0, R), :], s1buf, sem)
        pltpu.async_copy(lbl_hbm.at[b, d, pl.ds(h0, R), :], lblbuf, sem)
        pltpu.async_copy(lw_hbm.at[b, pl.ds(off, CH)], lwbuf, sem)

    def wait_set(si):
        lbuf, s0buf, s1buf, lblbuf, lwbuf = bufs[si]
        sem = sems[si]
        pltpu.make_async_copy(
            out_hbm.at[0, :, 0, pl.ds(0, R), :], lbuf, sem).wait()
        pltpu.make_async_copy(
            sfp_hbm.at[0, 0, 0, pl.ds(0, R), :], s0buf, sem).wait()
        pltpu.make_async_copy(
            sfp_hbm.at[0, 1, 0, pl.ds(0, R), :], s1buf, sem).wait()
        pltpu.make_async_copy(
            lbl_hbm.at[0, 0, pl.ds(0, R), :], lblbuf, sem).wait()
        pltpu.make_async_copy(lw_hbm.at[0, pl.ds(0, CH)], lwbuf, sem).wait()

    def process(si, acc, tot):
        lbuf, s0buf, s1buf, lblbuf, lwbuf = bufs[si]
        ccnt = plsc.parallel_loop(
            0, nv, 1, unroll=4, carry=jnp.int32(0))(
                functools.partial(mbody, s0buf, s1buf, lwbuf))
        idxbuf[pl.ds(ccnt, L)] = jnp.zeros((L,), jnp.int32)
        nvec = (ccnt + (L - 1)) // L
        acc = plsc.parallel_loop(
            0, nvec, 1, unroll=2, carry=acc)(
                functools.partial(cbody, lbuf, lblbuf, ccnt))
        return acc, tot + ccnt

    fire(0, 0)

    def outer(g, carry):
        acc, tot = carry
        c0 = g * 2
        fire(c0 + 1, 1)
        wait_set(0)
        acc, tot = process(0, acc, tot)

        @pl.when(c0 + 2 < nch)
        def _():
            fire(c0 + 2, 0)

        wait_set(1)
        acc, tot = process(1, acc, tot)
        return acc, tot

    acc, tot = lax.fori_loop(0, nch // 2, outer, (acc, tot))

    accv[...] = acc
    cntv[...] = jnp.where(lane == 0, jnp.full((L,), tot, jnp.int32), 0)
    orow = wid // 4
    ocol = (wid - orow * 4) * L
    pltpu.sync_copy(accv, sum_out.at[orow, pl.ds(ocol, L)])
    pltpu.sync_copy(cntv, cnt_out.at[orow, pl.ds(ocol, L)])


def _tc_reduce_body(sum_ref, cnt_ref, loss_ref, flag_ref):
    s = jnp.sum(sum_ref[:, :4 * L])
    c = jnp.sum(cnt_ref[:, :4 * L])
    denom = jnp.maximum(c.astype(jnp.float32), 1.0)
    loss_ref[...] = jnp.where(c > 0, s / denom, 0.0)
    flag_ref[...] = c > 0


@jax.jit
def kernel(output, label, label_weight, sketch_from_pred, weight):
    B = output.shape[0]
    ncls = output.shape[1]
    D, H, W = output.shape[2], output.shape[3], output.shape[4]
    M = D * H * W
    HW = H * W
    N = B * M
    PW = N // NW
    R = CH // W

    mesh = plsc.VectorSubcoreMesh(core_axis_name="c", subcore_axis_name="s")
    sc = pl.kernel(
        functools.partial(_sc_body, ncls, M, HW, W, PW),
        out_type=(
            jax.ShapeDtypeStruct((8, 8 * L), jnp.float32),
            jax.ShapeDtypeStruct((8, 8 * L), jnp.int32),
        ),
        mesh=mesh,
        compiler_params=pltpu.CompilerParams(
            use_tc_tiling_on_sc=False, needs_layout_passes=False),
        scratch_types=[
            pltpu.VMEM((ncls, R, W), jnp.float32),
            pltpu.VMEM((R, W), jnp.float32),
            pltpu.VMEM((R, W), jnp.float32),
            pltpu.VMEM((R, W), jnp.int32),
            pltpu.VMEM((CH,), jnp.int32),
            pltpu.VMEM((ncls, R, W), jnp.float32),
            pltpu.VMEM((R, W), jnp.float32),
            pltpu.VMEM((R, W), jnp.float32),
            pltpu.VMEM((R, W), jnp.int32),
            pltpu.VMEM((CH,), jnp.int32),
            pltpu.VMEM((CH + L,), jnp.int32),
            pltpu.VMEM((ncls,), jnp.float32),
            pltpu.VMEM((L,), jnp.float32),
            pltpu.VMEM((L,), jnp.int32),
            pltpu.SemaphoreType.DMA,
            pltpu.SemaphoreType.DMA,
        ],
    )
    sums, cnts = sc(output, label, label_weight, sketch_from_pred, weight)

    edge_loss, has_edges = pl.pallas_call(
        _tc_reduce_body,
        out_shape=(
            jax.ShapeDtypeStruct((), jnp.float32),
            jax.ShapeDtypeStruct((), jnp.bool_),
        ),
        out_specs=(
            pl.BlockSpec(memory_space=pltpu.SMEM),
            pl.BlockSpec(memory_space=pltpu.SMEM),
        ),
    )(sums, cnts)

    return (edge_loss, has_edges)
